# baseline (speedup 1.0000x reference)
import jax
import jax.numpy as jnp
from jax import lax
from jax.experimental import pallas as pl
from jax.experimental.pallas import tpu as pltpu
from jax.experimental.pallas import tpu_sc as plsc
from jax._src.pallas import mpmd as _mpmd

N_NODES = 1_000_000
MEM_DIM = 64
MSG_DIM = 128
B = 65536

NUM_WORKERS = 32
CHUNK = B // NUM_WORKERS
NSUB = 4
SUB = CHUNK // NSUB

def _matmul_body(msg_ref, w_ref, b_ref, out_ref):
    out_ref[...] = (
        jnp.dot(msg_ref[...], w_ref[...], preferred_element_type=jnp.float32)
        + b_ref[...]
    )


def _sc_scatter_body(ids_hbm, lo_hbm, ts_hbm, upd_hbm, mem0_hbm, lu0_hbm,
                     out_mem_hbm, out_lu_hbm,
                     idx_v, lo_v, rows_v, tse_v, sgm, sgt, ssm, sst):
    wid = lax.axis_index("s") * 2 + lax.axis_index("c")
    pltpu.sync_copy(ids_hbm.at[wid], idx_v)
    pltpu.sync_copy(lo_hbm.at[wid], lo_v)

    def start_gather(j):
        b = j % 2
        return (
            pltpu.async_copy(upd_hbm.at[lo_v.at[j]], rows_v.at[b], sgm),
            pltpu.async_copy(ts_hbm.at[lo_v.at[j]], tse_v.at[b], sgt),
        )

    g = start_gather(0)
    prev_s = None
    for j in range(NSUB):
        b = j % 2
        g[0].wait()
        g[1].wait()
        if j + 1 < NSUB and prev_s is not None:
            prev_s[0].wait()
            prev_s[1].wait()
        s = (
            pltpu.async_copy(rows_v.at[b], out_mem_hbm.at[idx_v.at[j]], ssm),
            pltpu.async_copy(tse_v.at[b], out_lu_hbm.at[idx_v.at[j]], sst),
        )
        if j + 1 < NSUB:
            g = start_gather(j + 1)
        prev_s, s = s, None
    prev_s[0].wait()
    prev_s[1].wait()


def _tc_matmul(messages, W, b):
    grid = B // CHUNK
    msg2 = messages.reshape(B // 2, 2 * MSG_DIM)
    zero = jnp.zeros_like(W)
    W2 = jnp.block([[W, zero], [zero, W]])
    b2 = jnp.concatenate([b, b]).reshape(1, 2 * MEM_DIM)
    return pl.pallas_call(
        _matmul_body,
        grid=(grid,),
        in_specs=[
            pl.BlockSpec((CHUNK // 2, 2 * MSG_DIM), lambda i: (i, 0)),
            pl.BlockSpec((2 * MSG_DIM, 2 * MEM_DIM), lambda i: (0, 0)),
            pl.BlockSpec((1, 2 * MEM_DIM), lambda i: (0, 0)),
        ],
        out_specs=pl.BlockSpec((CHUNK // 2, 2 * MEM_DIM), lambda i: (i, 0)),
        out_shape=jax.ShapeDtypeStruct((B // 2, 2 * MEM_DIM), jnp.float32),
    )(msg2, W2, b2)


def _sc_scatter(ids3, lo3, ts, updated, mem0, lu0):
    mesh = plsc.VectorSubcoreMesh(core_axis_name="c", subcore_axis_name="s")
    fn = _mpmd._mpmd_map(
        [(mesh, _sc_scatter_body)],
        [
            jax.ShapeDtypeStruct((N_NODES, MEM_DIM), jnp.float32),
            jax.ShapeDtypeStruct((N_NODES,), jnp.float32),
        ],
        input_output_aliases={4: 0, 5: 1},
        compiler_params=pltpu.CompilerParams(use_tc_tiling_on_sc=False),
        scratch_types=[
            pltpu.VMEM((NSUB, SUB), jnp.int32),
            pltpu.VMEM((NSUB, SUB), jnp.int32),
            pltpu.VMEM((2, SUB, MEM_DIM), jnp.float32),
            pltpu.VMEM((2, SUB), jnp.float32),
            pltpu.SemaphoreType.DMA,
            pltpu.SemaphoreType.DMA,
            pltpu.SemaphoreType.DMA,
            pltpu.SemaphoreType.DMA,
        ],
    )
    return fn(ids3, lo3, ts, updated, mem0, lu0)


def kernel(unique_node_ids, unique_messages, timestamps, memory, last_update,
           W, b):
    ids = unique_node_ids.astype(jnp.int32)
    iota = jnp.arange(B, dtype=jnp.int32)
    is_last = jnp.concatenate(
        [ids[1:] != ids[:-1], jnp.ones((1,), dtype=bool)])
    last_occ = lax.cummin(jnp.where(is_last, iota, B), axis=0, reverse=True)
    ids3 = ids.reshape(NUM_WORKERS, NSUB, SUB)
    lo3 = last_occ.reshape(NUM_WORKERS, NSUB, SUB)
    updated = _tc_matmul(unique_messages, W, b).reshape(B, MEM_DIM)
    out_mem, out_lu = _sc_scatter(ids3, lo3, timestamps, updated, memory,
                                  last_update)
    return (out_mem, out_lu)
